# Initial kernel scaffold; baseline (speedup 1.0000x reference)
import jax
import jax.numpy as jnp
from jax.experimental import pallas as pl


def kernel(x, edge_index, batch, W1, b1, g1, be1, W2, b2, g2, be2, Wc1, bc1, Wc2, bc2):
    raise NotImplementedError("write your pallas kernel here")


# speedup vs baseline: 20.8243x; 20.8243x over previous
import functools

import jax
import jax.numpy as jnp
from jax import lax
from jax.experimental import pallas as pl
from jax.experimental.pallas import tpu as pltpu
from jax.experimental.pallas import tpu_sc as plsc

N = 10000
E = 320000
D = 128
H = 256
HH = H // 2
G = 16
OUT = 2
EPS = 1e-5
NP = 10240
DW = 128

NC = 2
NS = 16
K = 80
KA = 40
ROWS_PER_SUB = NP // NS
EDGES_PER_SUB_DEG = E // (NC * NS)
EDGES_PER_SUB_AGG = E // NS

@functools.cache
def _mesh():
  return plsc.VectorSubcoreMesh(
      core_axis_name="c", subcore_axis_name="s", num_cores=NC, num_subcores=NS)

HIGHEST = lax.Precision.HIGHEST


NCHUNK_DEG = EDGES_PER_SUB_DEG // K
NCHUNK_AGG = EDGES_PER_SUB_AGG // KA


def _deg_body(dstr_hbm, ones_hbm, zeros_hbm, out_hbm, dst2d, ones_v, acc, sem):
  c = lax.axis_index("c")
  s = lax.axis_index("s")
  wid = c * NS + s
  rows = pl.ds(s * ROWS_PER_SUB, ROWS_PER_SUB)
  pltpu.sync_copy(ones_hbm, ones_v)
  pltpu.sync_copy(dstr_hbm.at[wid], dst2d)
  pltpu.sync_copy(zeros_hbm.at[rows], acc.at[rows])
  plsc.subcore_barrier()

  @pl.loop(0, NCHUNK_DEG // 5)
  def _(g):
    descs = [pltpu.async_copy(ones_v, acc.at[dst2d.at[g * 5 + i]], sem, add=True)
             for i in range(5)]
    for dsc in descs:
      dsc.wait()

  plsc.subcore_barrier()
  pltpu.sync_copy(acc.at[rows], out_hbm.at[c].at[rows])


@jax.jit
def _deg(dstr32, ones16, zeros16):
  return pl.kernel(
      _deg_body,
      out_type=jax.ShapeDtypeStruct((NC, NP, DW), jnp.float32),
      mesh=_mesh(),
      scratch_types=[
          pltpu.VMEM((NCHUNK_DEG, K), jnp.int32),
          pltpu.VMEM((K, DW), jnp.float32),
          pltpu.VMEM_SHARED((NP, DW), jnp.float32),
          pltpu.SemaphoreType.DMA,
      ],
  )(dstr32, ones16, zeros16)


NSEC = 5
CPS = NCHUNK_AGG // NSEC
QPS = CPS // 4
NBUF = 4


def _agg_body(h_hbm, srcr_hbm, dstr_hbm, out_hbm,
              src2d, dst2d, rows0, rows1, rows2, rows3, acc,
              sem0, sem1, sem2, sem3):
  c = lax.axis_index("c")
  s = lax.axis_index("s")
  rows = pl.ds(s * ROWS_PER_SUB, ROWS_PER_SUB)
  bufs = (rows0, rows1, rows2, rows3)
  sems = (sem0, sem1, sem2, sem3)
  pltpu.sync_copy(h_hbm.at[c].at[rows], acc.at[rows])
  plsc.subcore_barrier()

  dummy = h_hbm.at[c].at[pl.ds(0, KA)]

  for sec in range(NSEC):
    pltpu.sync_copy(srcr_hbm.at[s].at[sec], src2d)
    pltpu.sync_copy(dstr_hbm.at[s].at[sec], dst2d)
    for b in range(NBUF):
      pltpu.async_copy(h_hbm.at[c].at[src2d.at[b]], bufs[b], sems[b])

    @pl.loop(0, QPS)
    def _(q):
      j0 = NBUF * q
      for b in range(NBUF):
        j = j0 + b
        pltpu.make_async_copy(dummy, bufs[b], sems[b]).wait()
        pltpu.sync_copy(bufs[b], acc.at[dst2d.at[j]], add=True)

        @pl.when(q < QPS - 1)
        def _():
          pltpu.async_copy(h_hbm.at[c].at[src2d.at[j + NBUF]], bufs[b], sems[b])

  plsc.subcore_barrier()
  pltpu.sync_copy(acc.at[rows], out_hbm.at[c].at[rows])


@jax.jit
def _agg(hsplit, srcr, dstr):
  return pl.kernel(
      _agg_body,
      out_type=jax.ShapeDtypeStruct((NC, NP, HH), jnp.float32),
      mesh=_mesh(),
      scratch_types=[
          pltpu.VMEM((CPS, KA), jnp.int32),
          pltpu.VMEM((CPS, KA), jnp.int32),
          pltpu.VMEM((KA, HH), jnp.float32),
          pltpu.VMEM((KA, HH), jnp.float32),
          pltpu.VMEM((KA, HH), jnp.float32),
          pltpu.VMEM((KA, HH), jnp.float32),
          pltpu.VMEM_SHARED((NP, HH), jnp.float32),
          pltpu.SemaphoreType.DMA,
          pltpu.SemaphoreType.DMA,
          pltpu.SemaphoreType.DMA,
          pltpu.SemaphoreType.DMA,
      ],
  )(hsplit, srcr, dstr)


def _dinv_from_parts(degp_ref):
  deg = degp_ref[0][:N, 0:1] + degp_ref[1][:N, 0:1] + 1.0
  return lax.rsqrt(deg)


RB = 640


def _tc_a_body(x_ref, degp_ref, w1_ref, h_out_ref):
  deg = degp_ref[0][:, 0:1] + degp_ref[1][:, 0:1] + 1.0
  dinv = lax.rsqrt(deg)
  h = jnp.dot(x_ref[...], w1_ref[...],
              preferred_element_type=jnp.float32)
  hp = h * dinv
  h_out_ref[0] = hp[:, :HH]
  h_out_ref[1] = hp[:, HH:]


def _bn_relu(z, gamma, beta):
  mu = jnp.mean(z, axis=0, keepdims=True)
  var = jnp.mean((z - mu) * (z - mu), axis=0, keepdims=True)
  return jnp.maximum(gamma * (z - mu) * lax.rsqrt(var + EPS) + beta, 0.0)


def _tc_b_body(agg_ref, degp_ref, b1_ref, g1_ref, be1_ref, w2_ref,
               h_out_ref, stat_ref):
  p = pl.program_id(0)
  i = pl.program_id(1)
  deg = degp_ref[0][:, 0:1] + degp_ref[1][:, 0:1] + 1.0
  dinv = lax.rsqrt(deg)
  za = agg_ref[0] * dinv + b1_ref[:, :HH]
  zb = agg_ref[1] * dinv + b1_ref[:, HH:]
  z = jnp.concatenate([za, zb], axis=1)
  rows = i * RB + lax.broadcasted_iota(jnp.int32, (RB, 1), 0)
  valid = rows < N

  @pl.when(p == 0)
  def _():
    @pl.when(i == 0)
    def _():
      stat_ref[...] = jnp.zeros((2, H), jnp.float32)
    stat_ref[0:1] += jnp.sum(jnp.where(valid, z, 0.0), axis=0, keepdims=True)

  @pl.when(p == 1)
  def _():
    mu = stat_ref[0:1] * (1.0 / N)
    dev = z - mu
    stat_ref[1:2] += jnp.sum(jnp.where(valid, dev * dev, 0.0), axis=0,
                             keepdims=True)

  @pl.when(p == 2)
  def _():
    mu = stat_ref[0:1] * (1.0 / N)
    var = stat_ref[1:2] * (1.0 / N)
    y = jnp.maximum(
        g1_ref[...] * (z - mu) * lax.rsqrt(var + EPS) + be1_ref[...], 0.0)
    h2 = jnp.dot(y, w2_ref[...], preferred_element_type=jnp.float32)
    hp = h2 * dinv
    h_out_ref[0] = hp[:, :HH]
    h_out_ref[1] = hp[:, HH:]


@jax.jit
def _tc_b(aggsplit, degparts, b1, g1, be1, w2):
  return pl.pallas_call(
      _tc_b_body,
      grid=(3, NP // RB),
      in_specs=[
          pl.BlockSpec((NC, RB, HH), lambda p, i: (0, i, 0)),
          pl.BlockSpec((NC, RB, DW), lambda p, i: (0, i, 0)),
          pl.BlockSpec((1, H), lambda p, i: (0, 0)),
          pl.BlockSpec((1, H), lambda p, i: (0, 0)),
          pl.BlockSpec((1, H), lambda p, i: (0, 0)),
          pl.BlockSpec((H, H), lambda p, i: (0, 0)),
      ],
      out_specs=pl.BlockSpec((NC, RB, HH), lambda p, i: (0, i, 0)),
      out_shape=jax.ShapeDtypeStruct((NC, NP, HH), jnp.float32),
      scratch_shapes=[pltpu.VMEM((2, H), jnp.float32)],
  )(aggsplit, degparts, b1, g1, be1, w2)


def _tc_c_body(agg_ref, degp_ref, b2_ref, g2_ref, be2_ref, batch_ref,
               wc1_ref, bc1_ref, wc2_ref, bc2_ref, out_ref):
  dinv = _dinv_from_parts(degp_ref)
  za = agg_ref[0][:N] * dinv + b2_ref[:, :HH]
  zb = agg_ref[1][:N] * dinv + b2_ref[:, HH:]
  ya = _bn_relu(za, g2_ref[:, :HH], be2_ref[:, :HH])
  yb = _bn_relu(zb, g2_ref[:, HH:], be2_ref[:, HH:])
  gids = lax.broadcasted_iota(jnp.int32, (1, G), 1)
  onehot = (batch_ref[...] == gids).astype(jnp.float32)
  dn = (((0,), (0,)), ((), ()))
  sums_a = lax.dot_general(onehot, ya, dn,
                           preferred_element_type=jnp.float32,
                           precision=HIGHEST)
  sums_b = lax.dot_general(onehot, yb, dn,
                           preferred_element_type=jnp.float32,
                           precision=HIGHEST)
  cnts = jnp.sum(onehot, axis=0)[:, None]
  scale = 1.0 / jnp.maximum(cnts, 1.0)
  pooled = jnp.concatenate([sums_a * scale, sums_b * scale], axis=1)
  z = jnp.maximum(
      jnp.dot(pooled, wc1_ref[...], preferred_element_type=jnp.float32)
      + bc1_ref[...], 0.0)
  out_ref[...] = jnp.dot(z, wc2_ref[...],
                         preferred_element_type=jnp.float32) + bc2_ref[...]


@jax.jit
def _tc_c(aggsplit, degparts, b2, g2, be2, batch2d, wc1, bc1, wc2, bc2):
  return pl.pallas_call(
      _tc_c_body,
      out_shape=jax.ShapeDtypeStruct((G, OUT), jnp.float32),
  )(aggsplit, degparts, b2, g2, be2, batch2d, wc1, bc1, wc2, bc2)


def kernel(x, edge_index, batch, W1, b1, g1, be1, W2, b2, g2, be2,
           Wc1, bc1, Wc2, bc2):
  src = edge_index[0]
  dst = edge_index[1]
  srcr = src.reshape(NS, NSEC, CPS, KA)
  dstr = dst.reshape(NS, NSEC, CPS, KA)
  dstr32 = dst.reshape(NC * NS, NCHUNK_DEG, K)
  ones16 = jnp.ones((K, DW), jnp.float32)
  zeros16 = jnp.zeros((NP, DW), jnp.float32)

  degparts = _deg(dstr32, ones16, zeros16)
  xpad = jnp.concatenate([x, jnp.zeros((NP - N, D), jnp.float32)], axis=0)
  hsplit1 = _tc_a(xpad, degparts, W1)
  aggsplit1 = _agg(hsplit1, srcr, dstr)
  hsplit2 = _tc_b(aggsplit1, degparts, b1.reshape(1, H), g1.reshape(1, H),
                  be1.reshape(1, H), W2)
  aggsplit2 = _agg(hsplit2, srcr, dstr)
  logits = _tc_c(aggsplit2, degparts, b2.reshape(1, H), g2.reshape(1, H),
                 be2.reshape(1, H), batch.reshape(N, 1),
                 Wc1, bc1.reshape(1, H), Wc2, bc2.reshape(1, OUT))
  return logits
